# baseline (speedup 1.0000x reference)
import functools

import jax
import jax.numpy as jnp
from jax import lax
from jax.experimental import pallas as pl
from jax.experimental.pallas import tpu as pltpu
from jax.experimental.pallas import tpu_sc as plsc

N_FIELDS = 26
VOCAB = 1000
EMBED_DIM = 128
BATCH = 16384

NUM_WORKERS = 32
B_PER_W = BATCH // NUM_WORKERS
LANES = 16
GROUPS = B_PER_W // LANES


def _rowsum_body(t_ref, o_ref):
    x = t_ref[0]
    ones = jnp.ones((1, EMBED_DIM), dtype=jnp.float32)
    s = lax.dot_general(ones, x, (((1,), (1,)), ((), ())),
                        preferred_element_type=jnp.float32)
    o_ref[...] = s[None]


def _field_rowsums(tables):
    return pl.pallas_call(
        _rowsum_body,
        grid=(N_FIELDS,),
        in_specs=[pl.BlockSpec((1, VOCAB, EMBED_DIM), lambda i: (i, 0, 0))],
        out_specs=pl.BlockSpec((1, 1, VOCAB), lambda i: (i, 0, 0)),
        out_shape=jax.ShapeDtypeStruct((N_FIELDS, 1, VOCAB), jnp.float32),
    )(tables)


def _sc_gather_sum(rowsum_hbm, idx_hbm, bias_hbm, out_hbm,
                   rowsum_v, idx_v, out_v, bias_v):
    wid = lax.axis_index("s") * 2 + lax.axis_index("c")
    pltpu.sync_copy(rowsum_hbm, rowsum_v)
    pltpu.sync_copy(idx_hbm.at[pl.ds(wid * B_PER_W * N_FIELDS,
                                     B_PER_W * N_FIELDS)], idx_v)
    pltpu.sync_copy(bias_hbm, bias_v)
    bias_vec = bias_v[...]
    stride = lax.iota(jnp.int32, 16) * N_FIELDS

    def body(j, carry):
        acc = bias_vec
        base = stride + j * (LANES * N_FIELDS)
        for f in range(N_FIELDS):
            ids = plsc.load_gather(idx_v, [base + f])
            acc = acc + plsc.load_gather(rowsum_v, [ids + f * VOCAB])
        out_v[pl.ds(j * LANES, LANES)] = acc
        return carry

    lax.fori_loop(0, GROUPS, body, 0)
    pltpu.sync_copy(out_v, out_hbm.at[pl.ds(wid * B_PER_W, B_PER_W)])


_SC_KERNEL = functools.partial(
    pl.kernel,
    out_type=jax.ShapeDtypeStruct((BATCH,), jnp.float32),
    mesh=plsc.VectorSubcoreMesh(core_axis_name="c", subcore_axis_name="s"),
    compiler_params=pltpu.CompilerParams(needs_layout_passes=False),
    scratch_types=[
        pltpu.VMEM((N_FIELDS * VOCAB,), jnp.float32),
        pltpu.VMEM((N_FIELDS * B_PER_W,), jnp.int32),
        pltpu.VMEM((B_PER_W,), jnp.float32),
        pltpu.VMEM((LANES,), jnp.float32),
    ],
)(_sc_gather_sum)


def kernel(indices, tables, bias):
    rowsum = _field_rowsums(tables).reshape(N_FIELDS * VOCAB)
    flat_idx = indices.astype(jnp.int32).reshape(BATCH * N_FIELDS)
    bias16 = jnp.broadcast_to(bias.astype(jnp.float32), (LANES,))
    out_flat = _SC_KERNEL(rowsum, flat_idx, bias16)
    return out_flat.reshape(BATCH, 1)


# speedup vs baseline: 1.2372x; 1.2372x over previous
import functools

import jax
import jax.numpy as jnp
from jax import lax
from jax.experimental import pallas as pl
from jax.experimental.pallas import tpu as pltpu
from jax.experimental.pallas import tpu_sc as plsc

N_FIELDS = 26
VOCAB = 1000
EMBED_DIM = 128
BATCH = 16384

NUM_WORKERS = 32
B_PER_W = BATCH // NUM_WORKERS
LANES = 16
GROUPS = B_PER_W // LANES


def _rowsum_body(b_ref, t_ref, o_ref):
    x = t_ref[0]
    ones = jnp.ones((1, EMBED_DIM), dtype=jnp.float32)
    s = lax.dot_general(ones, x, (((1,), (1,)), ((), ())),
                        preferred_element_type=jnp.float32)
    s = jnp.where(pl.program_id(0) == 0, s + b_ref[0], s)
    o_ref[...] = s[None]


def _field_rowsums(tables, bias):
    return pl.pallas_call(
        _rowsum_body,
        grid=(N_FIELDS,),
        in_specs=[
            pl.BlockSpec(memory_space=pltpu.SMEM),
            pl.BlockSpec((1, VOCAB, EMBED_DIM), lambda i: (i, 0, 0)),
        ],
        out_specs=pl.BlockSpec((1, 1, VOCAB), lambda i: (i, 0, 0)),
        out_shape=jax.ShapeDtypeStruct((N_FIELDS, 1, VOCAB), jnp.float32),
    )(bias, tables)


def _sc_gather_sum(rowsum_hbm, idx_hbm, out_hbm, rowsum_v, idx_v, out_v):
    wid = lax.axis_index("s") * 2 + lax.axis_index("c")
    pltpu.sync_copy(rowsum_hbm, rowsum_v)
    pltpu.sync_copy(idx_hbm.at[wid], idx_v)

    def body(j, carry):
        acc = jnp.zeros((LANES,), jnp.float32)
        for f in range(N_FIELDS):
            idx = idx_v[pl.ds(f * B_PER_W + j * LANES, LANES)]
            acc = acc + plsc.load_gather(rowsum_v, [idx])
        out_v[pl.ds(j * LANES, LANES)] = acc
        return carry

    lax.fori_loop(0, GROUPS, body, 0)
    pltpu.sync_copy(out_v, out_hbm.at[pl.ds(wid * B_PER_W, B_PER_W)])


_SC_KERNEL = functools.partial(
    pl.kernel,
    out_type=jax.ShapeDtypeStruct((BATCH,), jnp.float32),
    mesh=plsc.VectorSubcoreMesh(core_axis_name="c", subcore_axis_name="s"),
    compiler_params=pltpu.CompilerParams(needs_layout_passes=False),
    scratch_types=[
        pltpu.VMEM((N_FIELDS * VOCAB,), jnp.float32),
        pltpu.VMEM((N_FIELDS * B_PER_W,), jnp.int32),
        pltpu.VMEM((B_PER_W,), jnp.float32),
    ],
)(_sc_gather_sum)


def kernel(indices, tables, bias):
    rowsum = _field_rowsums(tables, bias.astype(jnp.float32))
    rowsum = rowsum.reshape(N_FIELDS * VOCAB)
    flat = indices.astype(jnp.int32) + (
        jnp.arange(N_FIELDS, dtype=jnp.int32) * VOCAB)[None, :]
    idx_prep = (flat.T.reshape(N_FIELDS, NUM_WORKERS, B_PER_W)
                .transpose(1, 0, 2).reshape(NUM_WORKERS, N_FIELDS * B_PER_W))
    out_flat = _SC_KERNEL(rowsum, idx_prep)
    return out_flat.reshape(BATCH, 1)


# speedup vs baseline: 1.6157x; 1.3060x over previous
import functools

import jax
import jax.numpy as jnp
from jax import lax
from jax.experimental import pallas as pl
from jax.experimental.pallas import tpu as pltpu
from jax.experimental.pallas import tpu_sc as plsc

N_FIELDS = 26
VOCAB = 1000
EMBED_DIM = 128
BATCH = 16384

NUM_WORKERS = 32
B_PER_W = BATCH // NUM_WORKERS
LANES = 16
GROUPS = B_PER_W // LANES


FIELDS_PER_STEP = 13
TC_STEPS = N_FIELDS // FIELDS_PER_STEP


def _rowsum_body(b_ref, t_ref, o_ref):
    ones = jnp.ones((1, EMBED_DIM), dtype=jnp.float32)
    for f in range(FIELDS_PER_STEP):
        x = t_ref[f]
        s = lax.dot_general(ones, x, (((1,), (1,)), ((), ())),
                            preferred_element_type=jnp.float32)
        if f == 0:
            s = jnp.where(pl.program_id(0) == 0, s + b_ref[0], s)
        o_ref[f] = s


def _field_rowsums(tables, bias):
    return pl.pallas_call(
        _rowsum_body,
        grid=(TC_STEPS,),
        in_specs=[
            pl.BlockSpec(memory_space=pltpu.SMEM),
            pl.BlockSpec((FIELDS_PER_STEP, VOCAB, EMBED_DIM),
                         lambda i: (i, 0, 0)),
        ],
        out_specs=pl.BlockSpec((FIELDS_PER_STEP, 1, VOCAB),
                               lambda i: (i, 0, 0)),
        out_shape=jax.ShapeDtypeStruct((N_FIELDS, 1, VOCAB), jnp.float32),
    )(bias, tables)


def _sc_gather_sum(rowsum_hbm, idx_hbm, out_hbm, rowsum_v, idx_v, out_v):
    wid = lax.axis_index("s") * 2 + lax.axis_index("c")
    pltpu.sync_copy(rowsum_hbm, rowsum_v)
    pltpu.sync_copy(idx_hbm.at[wid], idx_v)

    def body(j, carry):
        acc = jnp.zeros((LANES,), jnp.float32)
        for f in range(N_FIELDS):
            idx = idx_v[pl.ds(f * B_PER_W + j * LANES, LANES)]
            acc = acc + plsc.load_gather(rowsum_v, [idx])
        out_v[pl.ds(j * LANES, LANES)] = acc
        return carry

    lax.fori_loop(0, GROUPS, body, 0)
    pltpu.sync_copy(out_v, out_hbm.at[pl.ds(wid * B_PER_W, B_PER_W)])


_SC_KERNEL = functools.partial(
    pl.kernel,
    out_type=jax.ShapeDtypeStruct((BATCH,), jnp.float32),
    mesh=plsc.VectorSubcoreMesh(core_axis_name="c", subcore_axis_name="s"),
    compiler_params=pltpu.CompilerParams(needs_layout_passes=False),
    scratch_types=[
        pltpu.VMEM((N_FIELDS * VOCAB,), jnp.float32),
        pltpu.VMEM((N_FIELDS * B_PER_W,), jnp.int32),
        pltpu.VMEM((B_PER_W,), jnp.float32),
    ],
)(_sc_gather_sum)


def kernel(indices, tables, bias):
    rowsum = _field_rowsums(tables, bias.astype(jnp.float32))
    rowsum = rowsum.reshape(N_FIELDS * VOCAB)
    flat = indices.astype(jnp.int32) + (
        jnp.arange(N_FIELDS, dtype=jnp.int32) * VOCAB)[None, :]
    idx_prep = (flat.T.reshape(N_FIELDS, NUM_WORKERS, B_PER_W)
                .transpose(1, 0, 2).reshape(NUM_WORKERS, N_FIELDS * B_PER_W))
    out_flat = _SC_KERNEL(rowsum, idx_prep)
    return out_flat.reshape(BATCH, 1)
